# baseline (speedup 1.0000x reference)
import functools

import jax
import jax.numpy as jnp
from jax import lax
from jax.experimental import pallas as pl
from jax.experimental.pallas import tpu as pltpu
from jax.experimental.pallas import tpu_sc as plsc

N = 100000
D = 128
B = 1024

_NC = 2
_NS = 16
_NW = _NC * _NS
_L = 16

_ROWS = 24576
_SPLIT = 2 * _ROWS
_CHUNK_B = N - _SPLIT
_PW_A = _SPLIT // _NW
_PW_B = 1600
_PAD_B = _NW * _PW_B


def _make_sc_gather(chunk_base, per_worker, out_pad):
    chunk_rows = {0: _SPLIT, _SPLIT: _CHUNK_B}[chunk_base]
    tail = chunk_base + chunk_rows - per_worker
    mesh = plsc.VectorSubcoreMesh(core_axis_name="c", subcore_axis_name="s")

    @functools.partial(
        pl.kernel,
        out_type=jax.ShapeDtypeStruct((out_pad,), jnp.float32),
        mesh=mesh,
        scratch_types=[
            pltpu.VMEM((B,), jnp.float32),
            pltpu.VMEM((per_worker,), jnp.int32),
            pltpu.VMEM((per_worker,), jnp.float32),
        ],
        compiler_params=pltpu.CompilerParams(needs_layout_passes=False),
    )
    def sc_gather(g_hbm, idx_hbm, out_hbm, g_v, idx_v, out_v):
        wid = lax.axis_index("s") * _NC + lax.axis_index("c")
        base = jnp.minimum(chunk_base + wid * per_worker, tail)
        pltpu.sync_copy(g_hbm, g_v)
        pltpu.sync_copy(idx_hbm.at[pl.ds(base, per_worker)], idx_v)

        @plsc.parallel_loop(0, per_worker, _L, unroll=8)
        def body(i):
            sl = pl.ds(i, _L)
            out_v[sl] = plsc.load_gather(g_v, [idx_v[sl]])

        pltpu.sync_copy(out_v, out_hbm.at[pl.ds(base - chunk_base, per_worker)])

    return sc_gather


@functools.cache
def _sc_gathers():
    return (_make_sc_gather(0, _PW_A, _SPLIT),
            _make_sc_gather(_SPLIT, _PW_B, _PAD_B))


def _expand_film(g_tile, s_ref, t_ref, z_ref):
    nsl = _ROWS // D
    g3 = jax.lax.broadcast_in_dim(g_tile, (nsl, D, D), (0, 1))
    z3 = z_ref[...].reshape(nsl, D, D)
    s3 = s_ref[...].reshape(1, 1, D)
    t3 = t_ref[...].reshape(1, 1, D)
    return (z3 * (1.0 + g3 * s3) + g3 * t3).reshape(_ROWS, D)


def _film_a_body(g_ref, s_ref, t_ref, z_ref, o_ref):
    o_ref[...] = _expand_film(g_ref[...], s_ref, t_ref, z_ref)


def _film_b_body(o_prev_ref, g_ref, s_ref, t_ref, z_ref, o_ref):
    del o_prev_ref
    o_ref[...] = _expand_film(g_ref[...], s_ref, t_ref, z_ref)


_GR = _ROWS // D


def kernel(z, g_graph, batch, s, t):
    idx = batch.astype(jnp.int32)
    sc_a, sc_b = _sc_gathers()
    g_a = sc_a(g_graph, idx).reshape(_SPLIT // D, D)
    g_b = sc_b(g_graph, idx).reshape(_PAD_B // D, D)
    s2 = s.reshape(1, D)
    t2 = t.reshape(1, D)

    out_a = pl.pallas_call(
        _film_a_body,
        out_shape=jax.ShapeDtypeStruct((N, D), jnp.float32),
        grid=(_SPLIT // _ROWS,),
        in_specs=[
            pl.BlockSpec((_GR, D), lambda i: (i, 0)),
            pl.BlockSpec((1, D), lambda i: (0, 0)),
            pl.BlockSpec((1, D), lambda i: (0, 0)),
            pl.BlockSpec((_ROWS, D), lambda i: (i, 0)),
        ],
        out_specs=pl.BlockSpec((_ROWS, D), lambda i: (i, 0)),
    )(g_a, s2, t2, z)

    nb = pl.cdiv(_CHUNK_B, _ROWS)
    off = _SPLIT // _ROWS
    return pl.pallas_call(
        _film_b_body,
        out_shape=jax.ShapeDtypeStruct((N, D), jnp.float32),
        grid=(nb,),
        in_specs=[
            pl.BlockSpec(memory_space=pl.ANY),
            pl.BlockSpec((_GR, D), lambda i: (i, 0)),
            pl.BlockSpec((1, D), lambda i: (0, 0)),
            pl.BlockSpec((1, D), lambda i: (0, 0)),
            pl.BlockSpec((_ROWS, D), lambda i: (i + off, 0)),
        ],
        out_specs=pl.BlockSpec((_ROWS, D), lambda i: (i + off, 0)),
        input_output_aliases={0: 0},
    )(out_a, g_b, s2, t2, z)


# speedup vs baseline: 1.0512x; 1.0512x over previous
import functools

import jax
import jax.numpy as jnp
from jax import lax
from jax.experimental import pallas as pl
from jax.experimental.pallas import tpu as pltpu
from jax.experimental.pallas import tpu_sc as plsc

N = 100000
D = 128
B = 1024

_NC = 2
_NS = 16
_NW = _NC * _NS
_NPW = 3136
_NPAD = _NW * _NPW
_TAIL = N - _NPW
_L = 16


@functools.cache
def _make_sc_gather():
    mesh = plsc.VectorSubcoreMesh(core_axis_name="c", subcore_axis_name="s")

    @functools.partial(
        pl.kernel,
        out_type=jax.ShapeDtypeStruct((_NPAD,), jnp.float32),
        mesh=mesh,
        scratch_types=[
            pltpu.VMEM((B,), jnp.float32),
            pltpu.VMEM((_NPW,), jnp.int32),
            pltpu.VMEM((_NPW,), jnp.float32),
        ],
        compiler_params=pltpu.CompilerParams(needs_layout_passes=False),
    )
    def sc_gather(g_hbm, idx_hbm, out_hbm, g_v, idx_v, out_v):
        wid = lax.axis_index("s") * _NC + lax.axis_index("c")
        base = jnp.where(wid == _NW - 1, _TAIL, wid * _NPW)
        pltpu.sync_copy(g_hbm, g_v)
        pltpu.sync_copy(idx_hbm.at[pl.ds(base, _NPW)], idx_v)

        @plsc.parallel_loop(0, _NPW, _L, unroll=8)
        def body(i):
            sl = pl.ds(i, _L)
            out_v[sl] = plsc.load_gather(g_v, [idx_v[sl]])
        pltpu.sync_copy(out_v, out_hbm.at[pl.ds(base, _NPW)])

    return sc_gather


_ROWS = 24576


def _film_body(g_ref, s_ref, t_ref, z_ref, o_ref):
    nsl = _ROWS // D
    g3 = jax.lax.broadcast_in_dim(g_ref[...], (nsl, D, D), (0, 1))
    z3 = z_ref[...].reshape(nsl, D, D)
    s3 = s_ref[...].reshape(1, 1, D)
    t3 = t_ref[...].reshape(1, 1, D)
    o_ref[...] = (z3 * (1.0 + g3 * s3) + g3 * t3).reshape(_ROWS, D)


def kernel(z, g_graph, batch, s, t):
    idx = batch.astype(jnp.int32)
    g_nodes = _make_sc_gather()(g_graph, idx)
    g3 = g_nodes.reshape(_NPAD // D, D)
    return pl.pallas_call(
        _film_body,
        out_shape=jax.ShapeDtypeStruct((N, D), jnp.float32),
        grid=(pl.cdiv(N, _ROWS),),
        in_specs=[
            pl.BlockSpec((_ROWS // D, D), lambda i: (i, 0)),
            pl.BlockSpec((1, D), lambda i: (0, 0)),
            pl.BlockSpec((1, D), lambda i: (0, 0)),
            pl.BlockSpec((_ROWS, D), lambda i: (i, 0)),
        ],
        out_specs=pl.BlockSpec((_ROWS, D), lambda i: (i, 0)),
    )(g3, s.reshape(1, D), t.reshape(1, D), z)


# speedup vs baseline: 1.0523x; 1.0011x over previous
import functools

import jax
import jax.numpy as jnp
from jax import lax
from jax.experimental import pallas as pl
from jax.experimental.pallas import tpu as pltpu
from jax.experimental.pallas import tpu_sc as plsc

N = 100000
D = 128
B = 1024

_NC = 2
_NS = 16
_NW = _NC * _NS
_NPW = 3136
_NPAD = _NW * _NPW
_TAIL = N - _NPW
_L = 16


@functools.cache
def _make_sc_gather():
    mesh = plsc.VectorSubcoreMesh(core_axis_name="c", subcore_axis_name="s")

    @functools.partial(
        pl.kernel,
        out_type=jax.ShapeDtypeStruct((_NPAD,), jnp.float32),
        mesh=mesh,
        scratch_types=[
            pltpu.VMEM((B,), jnp.float32),
            pltpu.VMEM((_NPW,), jnp.int32),
            pltpu.VMEM((_NPW,), jnp.float32),
        ],
        compiler_params=pltpu.CompilerParams(needs_layout_passes=False),
    )
    def sc_gather(g_hbm, idx_hbm, out_hbm, g_v, idx_v, out_v):
        wid = lax.axis_index("s") * _NC + lax.axis_index("c")
        base = jnp.where(wid == _NW - 1, _TAIL, wid * _NPW)
        pltpu.sync_copy(g_hbm, g_v)
        pltpu.sync_copy(idx_hbm.at[pl.ds(base, _NPW)], idx_v)

        @plsc.parallel_loop(0, _NPW, _L, unroll=4)
        def body(i):
            sl = pl.ds(i, _L)
            out_v[sl] = plsc.load_gather(g_v, [idx_v[sl]])
        pltpu.sync_copy(out_v, out_hbm.at[pl.ds(base, _NPW)])

    return sc_gather


_ROWS = 24576


def _film_body(g_ref, s_ref, t_ref, z_ref, o_ref):
    nsl = _ROWS // D
    g3 = jax.lax.broadcast_in_dim(g_ref[...], (nsl, D, D), (0, 1))
    z3 = z_ref[...].reshape(nsl, D, D)
    s3 = s_ref[...].reshape(1, 1, D)
    t3 = t_ref[...].reshape(1, 1, D)
    o_ref[...] = (z3 * (1.0 + g3 * s3) + g3 * t3).reshape(_ROWS, D)


def kernel(z, g_graph, batch, s, t):
    idx = batch.astype(jnp.int32)
    g_nodes = _make_sc_gather()(g_graph, idx)
    g3 = g_nodes.reshape(_NPAD // D, D)
    return pl.pallas_call(
        _film_body,
        out_shape=jax.ShapeDtypeStruct((N, D), jnp.float32),
        grid=(pl.cdiv(N, _ROWS),),
        in_specs=[
            pl.BlockSpec((_ROWS // D, D), lambda i: (i, 0)),
            pl.BlockSpec((1, D), lambda i: (0, 0)),
            pl.BlockSpec((1, D), lambda i: (0, 0)),
            pl.BlockSpec((_ROWS, D), lambda i: (i, 0)),
        ],
        out_specs=pl.BlockSpec((_ROWS, D), lambda i: (i, 0)),
    )(g3, s.reshape(1, D), t.reshape(1, D), z)


# speedup vs baseline: 1.0591x; 1.0064x over previous
import functools

import jax
import jax.numpy as jnp
from jax import lax
from jax.experimental import pallas as pl
from jax.experimental.pallas import tpu as pltpu
from jax.experimental.pallas import tpu_sc as plsc

N = 100000
D = 128
B = 1024

_NC = 2
_NS = 16
_NW = _NC * _NS
_NPW = 3136
_NPAD = _NW * _NPW
_TAIL = N - _NPW
_L = 16


@functools.cache
def _make_sc_gather():
    mesh = plsc.VectorSubcoreMesh(core_axis_name="c", subcore_axis_name="s")

    @functools.partial(
        pl.kernel,
        out_type=jax.ShapeDtypeStruct((_NPAD,), jnp.float32),
        mesh=mesh,
        scratch_types=[
            pltpu.VMEM((B,), jnp.float32),
            pltpu.VMEM((_NPW,), jnp.int32),
            pltpu.VMEM((_NPW,), jnp.float32),
            pltpu.SemaphoreType.DMA,
            pltpu.SemaphoreType.DMA,
        ],
        compiler_params=pltpu.CompilerParams(needs_layout_passes=False),
    )
    def sc_gather(g_hbm, idx_hbm, out_hbm, g_v, idx_v, out_v, sem_g, sem_i):
        wid = lax.axis_index("s") * _NC + lax.axis_index("c")
        base = jnp.where(wid == _NW - 1, _TAIL, wid * _NPW)
        cp_g = pltpu.async_copy(g_hbm, g_v, sem_g)
        cp_i = pltpu.async_copy(idx_hbm.at[pl.ds(base, _NPW)], idx_v, sem_i)
        cp_g.wait()
        cp_i.wait()

        @plsc.parallel_loop(0, _NPW, _L, unroll=4)
        def body(i):
            sl = pl.ds(i, _L)
            out_v[sl] = plsc.load_gather(g_v, [idx_v[sl]])
        pltpu.sync_copy(out_v, out_hbm.at[pl.ds(base, _NPW)])

    return sc_gather


_ROWS = 24576


def _film_body(g_ref, s_ref, t_ref, z_ref, o_ref):
    nsl = _ROWS // D
    g3 = jax.lax.broadcast_in_dim(g_ref[...], (nsl, D, D), (0, 1))
    z3 = z_ref[...].reshape(nsl, D, D)
    s3 = s_ref[...].reshape(1, 1, D)
    t3 = t_ref[...].reshape(1, 1, D)
    o_ref[...] = (z3 * (1.0 + g3 * s3) + g3 * t3).reshape(_ROWS, D)


def kernel(z, g_graph, batch, s, t):
    idx = batch.astype(jnp.int32)
    g_nodes = _make_sc_gather()(g_graph, idx)
    g3 = g_nodes.reshape(_NPAD // D, D)
    return pl.pallas_call(
        _film_body,
        out_shape=jax.ShapeDtypeStruct((N, D), jnp.float32),
        grid=(pl.cdiv(N, _ROWS),),
        in_specs=[
            pl.BlockSpec((_ROWS // D, D), lambda i: (i, 0)),
            pl.BlockSpec((1, D), lambda i: (0, 0)),
            pl.BlockSpec((1, D), lambda i: (0, 0)),
            pl.BlockSpec((_ROWS, D), lambda i: (i, 0)),
        ],
        out_specs=pl.BlockSpec((_ROWS, D), lambda i: (i, 0)),
    )(g3, s.reshape(1, D), t.reshape(1, D), z)
